# baseline (speedup 1.0000x reference)
import functools

import jax
import jax.numpy as jnp
from jax import lax
from jax.experimental import pallas as pl
from jax.experimental.pallas import tpu as pltpu
from jax.experimental.pallas import tpu_sc as plsc

F32 = jnp.float32


def _bf16_bits(f):
    bits = lax.bitcast_convert_type(f, jnp.uint32)
    rnd = (bits >> 16) & jnp.uint32(1)
    return (bits + jnp.uint32(0x7FFF) + rnd) >> 16


def _k1_body(state_ref, b_ref, wcat_ref, uv_ref, b1cat_ref, pcat_ref):
    dn = (((1,), (1,)), ((), ()))
    pcat = (lax.dot_general(state_ref[...], wcat_ref[...], dn,
                            preferred_element_type=F32)
            + b_ref[...] * uv_ref[...] + b1cat_ref[...])
    lo = _bf16_bits(pcat[:, 0:64])
    hi = _bf16_bits(pcat[:, 64:128])
    pcat_ref[...] = lax.bitcast_convert_type((hi << 16) | lo, jnp.int32)


def _node_projections(state_prev, b, wcat, uv, b1cat):
    n = state_prev.shape[0]
    return pl.pallas_call(
        _k1_body,
        out_shape=jax.ShapeDtypeStruct((n, 64), jnp.int32),
    )(state_prev, b, wcat, uv, b1cat)


def _gather_ab(pcat, src3d, dst3d):
    nblk = src3d.shape[0]
    w = 128
    e = nblk * w
    mesh = plsc.VectorSubcoreMesh(core_axis_name="core",
                                  subcore_axis_name="subcore")

    @functools.partial(
        pl.kernel,
        out_type=(jax.ShapeDtypeStruct((e, 64), jnp.int32),
                  jax.ShapeDtypeStruct((e, 64), jnp.int32)),
        mesh=mesh,
        scratch_types=[pltpu.SemaphoreType.DMA, pltpu.SemaphoreType.DMA],
        compiler_params=pltpu.CompilerParams(use_tc_tiling_on_sc=False),
    )
    def k2(pcat_hbm, src_hbm, dst_hbm, a_hbm, b_hbm, sem1, sem2):
        def body(si_vmem, di_vmem, a_vmem, b_vmem):
            c1 = pltpu.async_copy(pcat_hbm.at[si_vmem.at[0, 0]], a_vmem, sem1)
            c2 = pltpu.async_copy(pcat_hbm.at[di_vmem.at[0, 0]], b_vmem, sem2)
            c1.wait()
            c2.wait()

        pltpu.emit_pipeline(
            body,
            grid=(nblk,),
            in_specs=[pl.BlockSpec((1, 1, w), lambda i: (i, 0, 0)),
                      pl.BlockSpec((1, 1, w), lambda i: (i, 0, 0))],
            out_specs=[pl.BlockSpec((w, 64), lambda i: (i, 0)),
                       pl.BlockSpec((w, 64), lambda i: (i, 0))],
            core_axis_name=("core", "subcore"),
            dimension_semantics=(pltpu.PARALLEL,),
        )(src_hbm, dst_hbm, a_hbm, b_hbm)

    return k2(pcat, src3d, dst3d)


def _k3_body(x1_ref, x2_ref, j_ref, wj_ref, w2_ref, b2_ref, w3_ref, b3_ref,
             out_ref):
    x1f = lax.bitcast_convert_type(x1_ref[...] << 16, F32)
    x2f = lax.bitcast_convert_type(x2_ref[...] & jnp.int32(-65536), F32)
    s = x1f + x2f
    wj = wj_ref[...]
    jadd = jnp.concatenate([j_ref[:, 0:1] * wj, j_ref[:, 1:2] * wj], axis=1)
    x = jnp.maximum(s + jadd, 0.0)
    dn = (((1,), (1,)), ((), ()))
    bp = x.shape[0]
    for k, xk in ((0, x[:, 0:64]), (1, x[:, 64:128])):
        y = lax.dot_general(xk, w2_ref[...], dn, preferred_element_type=F32)
        y = jnp.maximum(y + b2_ref[...], 0.0)
        out_ref[pl.ds(k * bp, bp), :] = (
            lax.dot_general(y, w3_ref[...], dn, preferred_element_type=F32)
            + b3_ref[...])


def _edge_mlp(x1r, x2r, j2, wj, w2, b2r, w3, b3r):
    ep = x1r.shape[0]
    bp = 1000
    grid = (ep // bp,)
    return pl.pallas_call(
        _k3_body,
        grid=grid,
        in_specs=[
            pl.BlockSpec((bp, 128), lambda i: (i, 0)),
            pl.BlockSpec((bp, 128), lambda i: (i, 0)),
            pl.BlockSpec((bp, 2), lambda i: (i, 0)),
            pl.BlockSpec((1, 64), lambda i: (0, 0)),
            pl.BlockSpec((64, 64), lambda i: (0, 0)),
            pl.BlockSpec((1, 64), lambda i: (0, 0)),
            pl.BlockSpec((128, 64), lambda i: (0, 0)),
            pl.BlockSpec((1, 128), lambda i: (0, 0)),
        ],
        out_specs=pl.BlockSpec((2 * bp, 128), lambda i: (i, 0)),
        out_shape=jax.ShapeDtypeStruct((2 * ep, 128), F32),
    )(x1r, x2r, j2, wj, w2, b2r, w3, b3r)


def _scatter_acc(r, dst3d, zeros_blk, n_pad):
    e = r.shape[0]
    w = 128
    nblk = e // w
    rows_per_tile = n_pad // 16
    mesh = plsc.VectorSubcoreMesh(core_axis_name="core",
                                  subcore_axis_name="subcore")

    @functools.partial(
        pl.kernel,
        out_type=jax.ShapeDtypeStruct((2, n_pad, 128), F32),
        mesh=mesh,
        scratch_types=[
            pltpu.VMEM_SHARED((n_pad, 128), F32),
        ],
    )
    def k4(r_hbm, dst_hbm, z_hbm, out_hbm, acc_sp):
        cid = lax.axis_index("core")
        sid = lax.axis_index("subcore")
        row0 = sid * rows_per_tile

        pltpu.sync_copy(z_hbm, acc_sp.at[pl.ds(row0, rows_per_tile)])
        plsc.subcore_barrier()

        def body(r_vmem, di_vmem):
            pltpu.sync_copy(r_vmem, acc_sp.at[di_vmem.at[0, 0]], add=True)

        pltpu.emit_pipeline(
            body,
            grid=(nblk,),
            in_specs=[pl.BlockSpec((w, 128), lambda i: (i, 0)),
                      pl.BlockSpec((1, 1, w), lambda i: (i, 0, 0))],
            out_specs=[],
            core_axis_name=("core", "subcore"),
            dimension_semantics=(pltpu.PARALLEL,),
        )(r_hbm, dst_hbm)

        plsc.subcore_barrier()
        pltpu.sync_copy(acc_sp.at[pl.ds(row0, rows_per_tile)],
                        out_hbm.at[cid, pl.ds(row0, rows_per_tile)])

    return k4(r, dst3d, zeros_blk)


def _k5_body(accs_ref, state_ref, wih_ref, whh_ref, bih_ref, bhh_ref,
             out_ref):
    x = accs_ref[0] + accs_ref[1]
    for k in range(2, accs_ref.shape[0]):
        x = x + accs_ref[k]
    h = state_ref[...]
    dn = (((1,), (1,)), ((), ()))
    gx = lax.dot_general(x, wih_ref[0], dn, preferred_element_type=F32) \
        + bih_ref[0]
    gh = lax.dot_general(h, whh_ref[0], dn, preferred_element_type=F32) \
        + bhh_ref[0]
    d = 128
    rg = jax.nn.sigmoid(gx[:, :d] + gh[:, :d])
    zg = jax.nn.sigmoid(gx[:, d:2 * d] + gh[:, d:2 * d])
    ng = jnp.tanh(gx[:, 2 * d:] + rg * gh[:, 2 * d:])
    out_ref[...] = (1.0 - zg) * ng + zg * h


def _gru_update(accs, state_prev, wih_s, whh_s, bih_s, bhh_s):
    n = state_prev.shape[0]
    nacc = accs.shape[0]
    bn = 1000
    half = n // 2
    bph = half // bn
    grid = (n // bn,)
    return pl.pallas_call(
        _k5_body,
        grid=grid,
        in_specs=[
            pl.BlockSpec((nacc, bn, 128), lambda i: (0, i, 0)),
            pl.BlockSpec((bn, 128), lambda i: (i, 0)),
            pl.BlockSpec((1, 384, 128), lambda i: (i // bph, 0, 0)),
            pl.BlockSpec((1, 384, 128), lambda i: (i // bph, 0, 0)),
            pl.BlockSpec((1, 1, 384), lambda i: (i // bph, 0, 0)),
            pl.BlockSpec((1, 1, 384), lambda i: (i // bph, 0, 0)),
        ],
        out_specs=pl.BlockSpec((bn, 128), lambda i: (i, 0)),
        out_shape=jax.ShapeDtypeStruct((n, 128), F32),
    )(accs, state_prev, wih_s, whh_s, bih_s, bhh_s)


def kernel(msg_node, J_msg, b, state_prev, idx_msg_edge, node_idx,
           node_idx_inv, W1, b1, W2, b2, W3, b3, Wih1, Whh1, bih1, bhh1,
           Wih2, Whh2, bih2, bhh2):
    n, h = state_prev.shape
    e = msg_node.shape[0]
    del idx_msg_edge, node_idx, node_idx_inv

    wcat = jnp.concatenate([W1[:, :h], W1[:, h + 4:2 * h + 4]], axis=0)
    u = (W1[:, h] - W1[:, h + 1]).reshape(1, 64)
    v = (W1[:, 2 * h + 5] - W1[:, 2 * h + 4]).reshape(1, 64)
    uv = jnp.concatenate([u, v], axis=1)
    wj = (W1[:, h + 2] - W1[:, h + 3]
          + W1[:, 2 * h + 7] - W1[:, 2 * h + 6]).reshape(1, 64)
    b1cat = jnp.concatenate([b1.reshape(1, 64), jnp.zeros((1, 64), F32)],
                            axis=1)
    b2r = b2.reshape(1, 64)
    b3r = b3.reshape(1, 128)
    wih_s = jnp.stack([Wih1, Wih2])
    whh_s = jnp.stack([Whh1, Whh2])
    bih_s = jnp.stack([bih1, bih2]).reshape(2, 1, 384)
    bhh_s = jnp.stack([bhh1, bhh2]).reshape(2, 1, 384)
    src3d = msg_node[:, 0].reshape(e // 128, 1, 128)
    dst3d = msg_node[:, 1].reshape(e // 128, 1, 128)
    dstp3d = (msg_node[:, 1].reshape(e // 2000, 1000, 2)
              .transpose(0, 2, 1).reshape(e // 128, 1, 128))
    n_pad = ((n + 1279) // 1280) * 1280
    zeros_blk = jnp.zeros((n_pad // 16, 128), F32)

    pcat = _node_projections(state_prev, b, wcat, uv, b1cat)

    nch = 2
    ec = e // nch
    j2 = J_msg.reshape(e // 2, 2)
    parts = []
    for c in range(nch):
        eb = c * ec // 128
        x1, x2 = _gather_ab(pcat, src3d[eb:eb + ec // 128],
                            dst3d[eb:eb + ec // 128])
        msg = _edge_mlp(x1.reshape(ec // 2, 128), x2.reshape(ec // 2, 128),
                        j2[c * ec // 2:(c + 1) * ec // 2], wj, W2, b2r,
                        W3, b3r)
        parts.append(_scatter_acc(msg, dstp3d[eb:eb + ec // 128],
                                  zeros_blk, n_pad))
    accs = jnp.concatenate(parts, axis=0)
    return _gru_update(accs, state_prev, wih_s, whh_s, bih_s, bhh_s)


# speedup vs baseline: 1.2030x; 1.2030x over previous
import functools

import jax
import jax.numpy as jnp
from jax import lax
from jax.experimental import pallas as pl
from jax.experimental.pallas import tpu as pltpu
from jax.experimental.pallas import tpu_sc as plsc

F32 = jnp.float32


def _bf16_bits(f):
    bits = lax.bitcast_convert_type(f, jnp.uint32)
    rnd = (bits >> 16) & jnp.uint32(1)
    return (bits + jnp.uint32(0x7FFF) + rnd) >> 16


def _k1_body(state_ref, b_ref, wcat_ref, uv_ref, b1cat_ref, pcat_ref):
    dn = (((1,), (1,)), ((), ()))
    pcat = (lax.dot_general(state_ref[...], wcat_ref[...], dn,
                            preferred_element_type=F32)
            + b_ref[...] * uv_ref[...] + b1cat_ref[...])
    lo = _bf16_bits(pcat[:, 0:64])
    hi = _bf16_bits(pcat[:, 64:128])
    pcat_ref[...] = lax.bitcast_convert_type((hi << 16) | lo, jnp.int32)


def _node_projections(state_prev, b, wcat, uv, b1cat):
    n = state_prev.shape[0]
    return pl.pallas_call(
        _k1_body,
        out_shape=jax.ShapeDtypeStruct((n, 64), jnp.int32),
    )(state_prev, b, wcat, uv, b1cat)


def _gather_fuse(pcat, src3d, dst3d):
    nblk = src3d.shape[0]
    w = 128
    e = nblk * w
    mesh = plsc.VectorSubcoreMesh(core_axis_name="core",
                                  subcore_axis_name="subcore")

    @functools.partial(
        pl.kernel,
        out_type=jax.ShapeDtypeStruct((e // 2, 128), F32),
        mesh=mesh,
        scratch_types=[pltpu.SemaphoreType.DMA, pltpu.SemaphoreType.DMA,
                       pltpu.VMEM((w, 64), jnp.int32),
                       pltpu.VMEM((w, 64), jnp.int32)],
        compiler_params=pltpu.CompilerParams(use_tc_tiling_on_sc=False),
    )
    def k2(pcat_hbm, src_hbm, dst_hbm, o_hbm, sem1, sem2, a_vmem, b_vmem):
        def body(si_vmem, di_vmem, o_vmem):
            c1 = pltpu.async_copy(pcat_hbm.at[si_vmem.at[0, 0]], a_vmem, sem1)
            c2 = pltpu.async_copy(pcat_hbm.at[di_vmem.at[0, 0]], b_vmem, sem2)
            c1.wait()
            c2.wait()

            @pl.loop(0, w // 2)
            def _(r):
                for q in range(8):
                    edge = 2 * r + (q // 4)
                    col = 16 * (q % 4)
                    wa = a_vmem[edge, pl.ds(col, 16)]
                    wb = b_vmem[edge, pl.ds(col, 16)]
                    lo = lax.bitcast_convert_type(wa << 16, F32)
                    hi = lax.bitcast_convert_type(
                        wb & jnp.int32(-65536), F32)
                    o_vmem[r, pl.ds(16 * q, 16)] = lo + hi

        pltpu.emit_pipeline(
            body,
            grid=(nblk,),
            in_specs=[pl.BlockSpec((1, 1, w), lambda i: (i, 0, 0)),
                      pl.BlockSpec((1, 1, w), lambda i: (i, 0, 0))],
            out_specs=[pl.BlockSpec((w // 2, 128), lambda i: (i, 0))],
            core_axis_name=("core", "subcore"),
            dimension_semantics=(pltpu.PARALLEL,),
        )(src_hbm, dst_hbm, o_hbm)

    return k2(pcat, src3d, dst3d)


def _k3_body(s_ref, j_ref, wj_ref, w2_ref, b2_ref, w3_ref, b3_ref,
             out_ref):
    s = s_ref[...]
    wj = wj_ref[...]
    jadd = jnp.concatenate([j_ref[:, 0:1] * wj, j_ref[:, 1:2] * wj], axis=1)
    x = jnp.maximum(s + jadd, 0.0)
    dn = (((1,), (1,)), ((), ()))
    bp = x.shape[0]
    for k, xk in ((0, x[:, 0:64]), (1, x[:, 64:128])):
        y = lax.dot_general(xk, w2_ref[...], dn, preferred_element_type=F32)
        y = jnp.maximum(y + b2_ref[...], 0.0)
        out_ref[pl.ds(k * bp, bp), :] = (
            lax.dot_general(y, w3_ref[...], dn, preferred_element_type=F32)
            + b3_ref[...])


def _edge_mlp(pre, j2, wj, w2, b2r, w3, b3r):
    ep = pre.shape[0]
    bp = 1000
    grid = (ep // bp,)
    return pl.pallas_call(
        _k3_body,
        grid=grid,
        in_specs=[
            pl.BlockSpec((bp, 128), lambda i: (i, 0)),
            pl.BlockSpec((bp, 2), lambda i: (i, 0)),
            pl.BlockSpec((1, 64), lambda i: (0, 0)),
            pl.BlockSpec((64, 64), lambda i: (0, 0)),
            pl.BlockSpec((1, 64), lambda i: (0, 0)),
            pl.BlockSpec((128, 64), lambda i: (0, 0)),
            pl.BlockSpec((1, 128), lambda i: (0, 0)),
        ],
        out_specs=pl.BlockSpec((2 * bp, 128), lambda i: (i, 0)),
        out_shape=jax.ShapeDtypeStruct((2 * ep, 128), F32),
    )(pre, j2, wj, w2, b2r, w3, b3r)


def _scatter_acc(r, dst3d, zeros_blk, n_pad):
    e = r.shape[0]
    w = 128
    nblk = e // w
    rows_per_tile = n_pad // 16
    mesh = plsc.VectorSubcoreMesh(core_axis_name="core",
                                  subcore_axis_name="subcore")

    @functools.partial(
        pl.kernel,
        out_type=jax.ShapeDtypeStruct((2, n_pad, 128), F32),
        mesh=mesh,
        scratch_types=[
            pltpu.VMEM_SHARED((n_pad, 128), F32),
        ],
    )
    def k4(r_hbm, dst_hbm, z_hbm, out_hbm, acc_sp):
        cid = lax.axis_index("core")
        sid = lax.axis_index("subcore")
        row0 = sid * rows_per_tile

        pltpu.sync_copy(z_hbm, acc_sp.at[pl.ds(row0, rows_per_tile)])
        plsc.subcore_barrier()

        def body(r_vmem, di_vmem):
            pltpu.sync_copy(r_vmem, acc_sp.at[di_vmem.at[0, 0]], add=True)

        pltpu.emit_pipeline(
            body,
            grid=(nblk,),
            in_specs=[pl.BlockSpec((w, 128), lambda i: (i, 0)),
                      pl.BlockSpec((1, 1, w), lambda i: (i, 0, 0))],
            out_specs=[],
            core_axis_name=("core", "subcore"),
            dimension_semantics=(pltpu.PARALLEL,),
        )(r_hbm, dst_hbm)

        plsc.subcore_barrier()
        pltpu.sync_copy(acc_sp.at[pl.ds(row0, rows_per_tile)],
                        out_hbm.at[cid, pl.ds(row0, rows_per_tile)])

    return k4(r, dst3d, zeros_blk)


def _k5_body(accs_ref, state_ref, wih_ref, whh_ref, bih_ref, bhh_ref,
             out_ref):
    x = accs_ref[0] + accs_ref[1]
    for k in range(2, accs_ref.shape[0]):
        x = x + accs_ref[k]
    h = state_ref[...]
    dn = (((1,), (1,)), ((), ()))
    gx = lax.dot_general(x, wih_ref[0], dn, preferred_element_type=F32) \
        + bih_ref[0]
    gh = lax.dot_general(h, whh_ref[0], dn, preferred_element_type=F32) \
        + bhh_ref[0]
    d = 128
    rg = jax.nn.sigmoid(gx[:, :d] + gh[:, :d])
    zg = jax.nn.sigmoid(gx[:, d:2 * d] + gh[:, d:2 * d])
    ng = jnp.tanh(gx[:, 2 * d:] + rg * gh[:, 2 * d:])
    out_ref[...] = (1.0 - zg) * ng + zg * h


def _gru_update(accs, state_prev, wih_s, whh_s, bih_s, bhh_s):
    n = state_prev.shape[0]
    nacc = accs.shape[0]
    bn = 1000
    half = n // 2
    bph = half // bn
    grid = (n // bn,)
    return pl.pallas_call(
        _k5_body,
        grid=grid,
        in_specs=[
            pl.BlockSpec((nacc, bn, 128), lambda i: (0, i, 0)),
            pl.BlockSpec((bn, 128), lambda i: (i, 0)),
            pl.BlockSpec((1, 384, 128), lambda i: (i // bph, 0, 0)),
            pl.BlockSpec((1, 384, 128), lambda i: (i // bph, 0, 0)),
            pl.BlockSpec((1, 1, 384), lambda i: (i // bph, 0, 0)),
            pl.BlockSpec((1, 1, 384), lambda i: (i // bph, 0, 0)),
        ],
        out_specs=pl.BlockSpec((bn, 128), lambda i: (i, 0)),
        out_shape=jax.ShapeDtypeStruct((n, 128), F32),
    )(accs, state_prev, wih_s, whh_s, bih_s, bhh_s)


def kernel(msg_node, J_msg, b, state_prev, idx_msg_edge, node_idx,
           node_idx_inv, W1, b1, W2, b2, W3, b3, Wih1, Whh1, bih1, bhh1,
           Wih2, Whh2, bih2, bhh2):
    n, h = state_prev.shape
    e = msg_node.shape[0]
    del idx_msg_edge, node_idx, node_idx_inv

    wcat = jnp.concatenate([W1[:, :h], W1[:, h + 4:2 * h + 4]], axis=0)
    u = (W1[:, h] - W1[:, h + 1]).reshape(1, 64)
    v = (W1[:, 2 * h + 5] - W1[:, 2 * h + 4]).reshape(1, 64)
    uv = jnp.concatenate([u, v], axis=1)
    wj = (W1[:, h + 2] - W1[:, h + 3]
          + W1[:, 2 * h + 7] - W1[:, 2 * h + 6]).reshape(1, 64)
    b1cat = jnp.concatenate([b1.reshape(1, 64), jnp.zeros((1, 64), F32)],
                            axis=1)
    b2r = b2.reshape(1, 64)
    b3r = b3.reshape(1, 128)
    wih_s = jnp.stack([Wih1, Wih2])
    whh_s = jnp.stack([Whh1, Whh2])
    bih_s = jnp.stack([bih1, bih2]).reshape(2, 1, 384)
    bhh_s = jnp.stack([bhh1, bhh2]).reshape(2, 1, 384)
    src3d = msg_node[:, 0].reshape(e // 128, 1, 128)
    dst3d = msg_node[:, 1].reshape(e // 128, 1, 128)
    dstp3d = (msg_node[:, 1].reshape(e // 2000, 1000, 2)
              .transpose(0, 2, 1).reshape(e // 128, 1, 128))
    n_pad = ((n + 1279) // 1280) * 1280
    zeros_blk = jnp.zeros((n_pad // 16, 128), F32)

    pcat = _node_projections(state_prev, b, wcat, uv, b1cat)
    pre = _gather_fuse(pcat, src3d, dst3d)
    msg = _edge_mlp(pre, J_msg.reshape(e // 2, 2), wj, W2, b2r, W3, b3r)
    accs = _scatter_acc(msg, dstp3d, zeros_blk, n_pad)
    return _gru_update(accs, state_prev, wih_s, whh_s, bih_s, bhh_s)


# speedup vs baseline: 1.2198x; 1.0140x over previous
import functools

import jax
import jax.numpy as jnp
from jax import lax
from jax.experimental import pallas as pl
from jax.experimental.pallas import tpu as pltpu
from jax.experimental.pallas import tpu_sc as plsc

F32 = jnp.float32


def _bf16_bits(f):
    bits = lax.bitcast_convert_type(f, jnp.uint32)
    rnd = (bits >> 16) & jnp.uint32(1)
    return (bits + jnp.uint32(0x7FFF) + rnd) >> 16


def _k1_body(state_ref, b_ref, wcat_ref, uv_ref, b1cat_ref, pcat_ref):
    dn = (((1,), (1,)), ((), ()))
    pcat = (lax.dot_general(state_ref[...], wcat_ref[...], dn,
                            preferred_element_type=F32)
            + b_ref[...] * uv_ref[...] + b1cat_ref[...])
    lo = _bf16_bits(pcat[:, 0:64])
    hi = _bf16_bits(pcat[:, 64:128])
    pcat_ref[...] = lax.bitcast_convert_type((hi << 16) | lo, jnp.int32)


def _node_projections(state_prev, b, wcat, uv, b1cat):
    n = state_prev.shape[0]
    return pl.pallas_call(
        _k1_body,
        out_shape=jax.ShapeDtypeStruct((n, 64), jnp.int32),
    )(state_prev, b, wcat, uv, b1cat)


def _gather_fuse(pcat, src3d, dst3d):
    nblk = src3d.shape[0]
    w = 128
    e = nblk * w
    mesh = plsc.VectorSubcoreMesh(core_axis_name="core",
                                  subcore_axis_name="subcore")

    @functools.partial(
        pl.kernel,
        out_type=jax.ShapeDtypeStruct((e // 2, 128), F32),
        mesh=mesh,
        scratch_types=[pltpu.SemaphoreType.DMA, pltpu.SemaphoreType.DMA,
                       pltpu.VMEM((w, 64), jnp.int32),
                       pltpu.VMEM((w, 64), jnp.int32)],
        compiler_params=pltpu.CompilerParams(use_tc_tiling_on_sc=False),
    )
    def k2(pcat_hbm, src_hbm, dst_hbm, o_hbm, sem1, sem2, a_vmem, b_vmem):
        def body(si_vmem, di_vmem, o_vmem):
            c1 = pltpu.async_copy(pcat_hbm.at[si_vmem.at[0, 0]], a_vmem, sem1)
            c2 = pltpu.async_copy(pcat_hbm.at[di_vmem.at[0, 0]], b_vmem, sem2)
            c1.wait()
            c2.wait()

            @plsc.parallel_loop(0, w // 2, unroll=4)
            def _(r):
                for q in range(8):
                    edge = 2 * r + (q // 4)
                    col = 16 * (q % 4)
                    wa = a_vmem[edge, pl.ds(col, 16)]
                    wb = b_vmem[edge, pl.ds(col, 16)]
                    lo = lax.bitcast_convert_type(wa << 16, F32)
                    hi = lax.bitcast_convert_type(
                        wb & jnp.int32(-65536), F32)
                    o_vmem[r, pl.ds(16 * q, 16)] = lo + hi

        pltpu.emit_pipeline(
            body,
            grid=(nblk,),
            in_specs=[pl.BlockSpec((1, 1, w), lambda i: (i, 0, 0)),
                      pl.BlockSpec((1, 1, w), lambda i: (i, 0, 0))],
            out_specs=[pl.BlockSpec((w // 2, 128), lambda i: (i, 0))],
            core_axis_name=("core", "subcore"),
            dimension_semantics=(pltpu.PARALLEL,),
        )(src_hbm, dst_hbm, o_hbm)

    return k2(pcat, src3d, dst3d)


def _k3_body(s_ref, j_ref, wj_ref, w2_ref, b2_ref, w3_ref, b3_ref,
             out_ref):
    s = s_ref[...]
    wj = wj_ref[...]
    jadd = jnp.concatenate([j_ref[:, 0:1] * wj, j_ref[:, 1:2] * wj], axis=1)
    x = jnp.maximum(s + jadd, 0.0)
    dn = (((1,), (1,)), ((), ()))
    bp = x.shape[0]
    for k, xk in ((0, x[:, 0:64]), (1, x[:, 64:128])):
        y = lax.dot_general(xk, w2_ref[...], dn, preferred_element_type=F32)
        y = jnp.maximum(y + b2_ref[...], 0.0)
        out_ref[pl.ds(k * bp, bp), :] = (
            lax.dot_general(y, w3_ref[...], dn, preferred_element_type=F32)
            + b3_ref[...])


def _edge_mlp(pre, j2, wj, w2, b2r, w3, b3r):
    ep = pre.shape[0]
    bp = 1000
    grid = (ep // bp,)
    return pl.pallas_call(
        _k3_body,
        grid=grid,
        in_specs=[
            pl.BlockSpec((bp, 128), lambda i: (i, 0)),
            pl.BlockSpec((bp, 2), lambda i: (i, 0)),
            pl.BlockSpec((1, 64), lambda i: (0, 0)),
            pl.BlockSpec((64, 64), lambda i: (0, 0)),
            pl.BlockSpec((1, 64), lambda i: (0, 0)),
            pl.BlockSpec((128, 64), lambda i: (0, 0)),
            pl.BlockSpec((1, 128), lambda i: (0, 0)),
        ],
        out_specs=pl.BlockSpec((2 * bp, 128), lambda i: (i, 0)),
        out_shape=jax.ShapeDtypeStruct((2 * ep, 128), F32),
    )(pre, j2, wj, w2, b2r, w3, b3r)


def _scatter_acc(r, dst3d, zeros_blk, n_pad):
    e = r.shape[0]
    w = 128
    nblk = e // w
    rows_per_tile = n_pad // 16
    mesh = plsc.VectorSubcoreMesh(core_axis_name="core",
                                  subcore_axis_name="subcore")

    @functools.partial(
        pl.kernel,
        out_type=jax.ShapeDtypeStruct((2, n_pad, 128), F32),
        mesh=mesh,
        scratch_types=[
            pltpu.VMEM_SHARED((n_pad, 128), F32),
        ],
    )
    def k4(r_hbm, dst_hbm, z_hbm, out_hbm, acc_sp):
        cid = lax.axis_index("core")
        sid = lax.axis_index("subcore")
        row0 = sid * rows_per_tile

        pltpu.sync_copy(z_hbm, acc_sp.at[pl.ds(row0, rows_per_tile)])
        plsc.subcore_barrier()

        def body(r_vmem, di_vmem):
            pltpu.sync_copy(r_vmem, acc_sp.at[di_vmem.at[0, 0]], add=True)

        pltpu.emit_pipeline(
            body,
            grid=(nblk,),
            in_specs=[pl.BlockSpec((w, 128), lambda i: (i, 0)),
                      pl.BlockSpec((1, 1, w), lambda i: (i, 0, 0))],
            out_specs=[],
            core_axis_name=("core", "subcore"),
            dimension_semantics=(pltpu.PARALLEL,),
        )(r_hbm, dst_hbm)

        plsc.subcore_barrier()
        pltpu.sync_copy(acc_sp.at[pl.ds(row0, rows_per_tile)],
                        out_hbm.at[cid, pl.ds(row0, rows_per_tile)])

    return k4(r, dst3d, zeros_blk)


def _k5_body(accs_ref, state_ref, wih_ref, whh_ref, bih_ref, bhh_ref,
             out_ref):
    x = accs_ref[0] + accs_ref[1]
    for k in range(2, accs_ref.shape[0]):
        x = x + accs_ref[k]
    h = state_ref[...]
    dn = (((1,), (1,)), ((), ()))
    gx = lax.dot_general(x, wih_ref[0], dn, preferred_element_type=F32) \
        + bih_ref[0]
    gh = lax.dot_general(h, whh_ref[0], dn, preferred_element_type=F32) \
        + bhh_ref[0]
    d = 128
    rg = jax.nn.sigmoid(gx[:, :d] + gh[:, :d])
    zg = jax.nn.sigmoid(gx[:, d:2 * d] + gh[:, d:2 * d])
    ng = jnp.tanh(gx[:, 2 * d:] + rg * gh[:, 2 * d:])
    out_ref[...] = (1.0 - zg) * ng + zg * h


def _gru_update(accs, state_prev, wih_s, whh_s, bih_s, bhh_s):
    n = state_prev.shape[0]
    nacc = accs.shape[0]
    bn = 1000
    half = n // 2
    bph = half // bn
    grid = (n // bn,)
    return pl.pallas_call(
        _k5_body,
        grid=grid,
        in_specs=[
            pl.BlockSpec((nacc, bn, 128), lambda i: (0, i, 0)),
            pl.BlockSpec((bn, 128), lambda i: (i, 0)),
            pl.BlockSpec((1, 384, 128), lambda i: (i // bph, 0, 0)),
            pl.BlockSpec((1, 384, 128), lambda i: (i // bph, 0, 0)),
            pl.BlockSpec((1, 1, 384), lambda i: (i // bph, 0, 0)),
            pl.BlockSpec((1, 1, 384), lambda i: (i // bph, 0, 0)),
        ],
        out_specs=pl.BlockSpec((bn, 128), lambda i: (i, 0)),
        out_shape=jax.ShapeDtypeStruct((n, 128), F32),
    )(accs, state_prev, wih_s, whh_s, bih_s, bhh_s)


def kernel(msg_node, J_msg, b, state_prev, idx_msg_edge, node_idx,
           node_idx_inv, W1, b1, W2, b2, W3, b3, Wih1, Whh1, bih1, bhh1,
           Wih2, Whh2, bih2, bhh2):
    n, h = state_prev.shape
    e = msg_node.shape[0]
    del idx_msg_edge, node_idx, node_idx_inv

    wcat = jnp.concatenate([W1[:, :h], W1[:, h + 4:2 * h + 4]], axis=0)
    u = (W1[:, h] - W1[:, h + 1]).reshape(1, 64)
    v = (W1[:, 2 * h + 5] - W1[:, 2 * h + 4]).reshape(1, 64)
    uv = jnp.concatenate([u, v], axis=1)
    wj = (W1[:, h + 2] - W1[:, h + 3]
          + W1[:, 2 * h + 7] - W1[:, 2 * h + 6]).reshape(1, 64)
    b1cat = jnp.concatenate([b1.reshape(1, 64), jnp.zeros((1, 64), F32)],
                            axis=1)
    b2r = b2.reshape(1, 64)
    b3r = b3.reshape(1, 128)
    wih_s = jnp.stack([Wih1, Wih2])
    whh_s = jnp.stack([Whh1, Whh2])
    bih_s = jnp.stack([bih1, bih2]).reshape(2, 1, 384)
    bhh_s = jnp.stack([bhh1, bhh2]).reshape(2, 1, 384)
    src3d = msg_node[:, 0].reshape(e // 128, 1, 128)
    dst3d = msg_node[:, 1].reshape(e // 128, 1, 128)
    dstp3d = (msg_node[:, 1].reshape(e // 2000, 1000, 2)
              .transpose(0, 2, 1).reshape(e // 128, 1, 128))
    n_pad = ((n + 1279) // 1280) * 1280
    zeros_blk = jnp.zeros((n_pad // 16, 128), F32)

    pcat = _node_projections(state_prev, b, wcat, uv, b1cat)
    pre = _gather_fuse(pcat, src3d, dst3d)
    msg = _edge_mlp(pre, J_msg.reshape(e // 2, 2), wj, W2, b2r, W3, b3r)
    accs = _scatter_acc(msg, dstp3d, zeros_blk, n_pad)
    return _gru_update(accs, state_prev, wih_s, whh_s, bih_s, bhh_s)


# speedup vs baseline: 1.2858x; 1.0541x over previous
import functools

import jax
import jax.numpy as jnp
from jax import lax
from jax.experimental import pallas as pl
from jax.experimental.pallas import tpu as pltpu
from jax.experimental.pallas import tpu_sc as plsc

F32 = jnp.float32


def _k1_body(state_ref, b_ref, wcat_ref, uv_ref, b1cat_ref, pcat_ref):
    dn = (((1,), (1,)), ((), ()))
    pcat_ref[...] = (lax.dot_general(state_ref[...], wcat_ref[...], dn,
                                     preferred_element_type=F32)
                     + b_ref[...] * uv_ref[...] + b1cat_ref[...])


def _node_projections(state_prev, b, wcat, uv, b1cat):
    n = state_prev.shape[0]
    return pl.pallas_call(
        _k1_body,
        out_shape=jax.ShapeDtypeStruct((n, 128), F32),
    )(state_prev, b, wcat, uv, b1cat)


def _gather_ab(pcat, src3d, dst3d):
    nblk = src3d.shape[0]
    w = 128
    e = nblk * w
    mesh = plsc.VectorSubcoreMesh(core_axis_name="core",
                                  subcore_axis_name="subcore")

    @functools.partial(
        pl.kernel,
        out_type=(jax.ShapeDtypeStruct((e, 128), F32),
                  jax.ShapeDtypeStruct((e, 128), F32)),
        mesh=mesh,
        scratch_types=[pltpu.SemaphoreType.DMA, pltpu.SemaphoreType.DMA],
    )
    def k2(pcat_hbm, src_hbm, dst_hbm, a_hbm, b_hbm, sem1, sem2):
        def body(si_vmem, di_vmem, a_vmem, b_vmem):
            c1 = pltpu.async_copy(pcat_hbm.at[si_vmem.at[0, 0]], a_vmem, sem1)
            c2 = pltpu.async_copy(pcat_hbm.at[di_vmem.at[0, 0]], b_vmem, sem2)
            c1.wait()
            c2.wait()

        pltpu.emit_pipeline(
            body,
            grid=(nblk,),
            in_specs=[pl.BlockSpec((1, 1, w), lambda i: (i, 0, 0)),
                      pl.BlockSpec((1, 1, w), lambda i: (i, 0, 0))],
            out_specs=[pl.BlockSpec((w, 128), lambda i: (i, 0)),
                       pl.BlockSpec((w, 128), lambda i: (i, 0))],
            core_axis_name=("core", "subcore"),
            dimension_semantics=(pltpu.PARALLEL,),
        )(src_hbm, dst_hbm, a_hbm, b_hbm)

    return k2(pcat, src3d, dst3d)


def _k3_body(x1_ref, x2_ref, j_ref, wj_ref, w2_ref, b2_ref, w3_ref, b3_ref,
             out_ref):
    dn = (((1,), (1,)), ((), ()))
    jterm = lax.dot_general(j_ref[...], wj_ref[...], (((1,), (0,)), ((), ())),
                            preferred_element_type=F32)
    x = jnp.maximum(x1_ref[:, 0:64] + x2_ref[:, 64:128] + jterm, 0.0)
    y = lax.dot_general(x, w2_ref[...], dn, preferred_element_type=F32)
    y = jnp.maximum(y + b2_ref[...], 0.0)
    out_ref[...] = (lax.dot_general(y, w3_ref[...], dn,
                                    preferred_element_type=F32)
                    + b3_ref[...])


def _edge_mlp(x1, x2, j_msg, wj, w2, b2r, w3, b3r):
    e = x1.shape[0]
    be = 2000
    grid = (e // be,)
    return pl.pallas_call(
        _k3_body,
        grid=grid,
        in_specs=[
            pl.BlockSpec((be, 128), lambda i: (i, 0)),
            pl.BlockSpec((be, 128), lambda i: (i, 0)),
            pl.BlockSpec((be, 1), lambda i: (i, 0)),
            pl.BlockSpec((1, 64), lambda i: (0, 0)),
            pl.BlockSpec((64, 64), lambda i: (0, 0)),
            pl.BlockSpec((1, 64), lambda i: (0, 0)),
            pl.BlockSpec((128, 64), lambda i: (0, 0)),
            pl.BlockSpec((1, 128), lambda i: (0, 0)),
        ],
        out_specs=pl.BlockSpec((be, 128), lambda i: (i, 0)),
        out_shape=jax.ShapeDtypeStruct((e, 128), F32),
    )(x1, x2, j_msg, wj, w2, b2r, w3, b3r)


def _scatter_acc(r, dst3d, zeros_blk, n_pad):
    e = r.shape[0]
    w = 128
    nblk = e // w
    rows_per_tile = n_pad // 16
    mesh = plsc.VectorSubcoreMesh(core_axis_name="core",
                                  subcore_axis_name="subcore")

    @functools.partial(
        pl.kernel,
        out_type=jax.ShapeDtypeStruct((2, n_pad, 128), F32),
        mesh=mesh,
        scratch_types=[
            pltpu.VMEM_SHARED((n_pad, 128), F32),
        ],
    )
    def k4(r_hbm, dst_hbm, z_hbm, out_hbm, acc_sp):
        cid = lax.axis_index("core")
        sid = lax.axis_index("subcore")
        row0 = sid * rows_per_tile

        pltpu.sync_copy(z_hbm, acc_sp.at[pl.ds(row0, rows_per_tile)])
        plsc.subcore_barrier()

        def body(r_vmem, di_vmem):
            pltpu.sync_copy(r_vmem, acc_sp.at[di_vmem.at[0, 0]], add=True)

        pltpu.emit_pipeline(
            body,
            grid=(nblk,),
            in_specs=[pl.BlockSpec((w, 128), lambda i: (i, 0)),
                      pl.BlockSpec((1, 1, w), lambda i: (i, 0, 0))],
            out_specs=[],
            core_axis_name=("core", "subcore"),
            dimension_semantics=(pltpu.PARALLEL,),
        )(r_hbm, dst_hbm)

        plsc.subcore_barrier()
        pltpu.sync_copy(acc_sp.at[pl.ds(row0, rows_per_tile)],
                        out_hbm.at[cid, pl.ds(row0, rows_per_tile)])

    return k4(r, dst3d, zeros_blk)


def _k5_body(accs_ref, state_ref, wih_ref, whh_ref, bih_ref, bhh_ref,
             out_ref):
    x = accs_ref[0] + accs_ref[1]
    for k in range(2, accs_ref.shape[0]):
        x = x + accs_ref[k]
    h = state_ref[...]
    dn = (((1,), (1,)), ((), ()))
    gx = lax.dot_general(x, wih_ref[0], dn, preferred_element_type=F32) \
        + bih_ref[0]
    gh = lax.dot_general(h, whh_ref[0], dn, preferred_element_type=F32) \
        + bhh_ref[0]
    d = 128
    rg = jax.nn.sigmoid(gx[:, :d] + gh[:, :d])
    zg = jax.nn.sigmoid(gx[:, d:2 * d] + gh[:, d:2 * d])
    ng = jnp.tanh(gx[:, 2 * d:] + rg * gh[:, 2 * d:])
    out_ref[...] = (1.0 - zg) * ng + zg * h


def _gru_update(accs, state_prev, wih_s, whh_s, bih_s, bhh_s):
    n = state_prev.shape[0]
    nacc = accs.shape[0]
    bn = 1000
    half = n // 2
    bph = half // bn
    grid = (n // bn,)
    return pl.pallas_call(
        _k5_body,
        grid=grid,
        in_specs=[
            pl.BlockSpec((nacc, bn, 128), lambda i: (0, i, 0)),
            pl.BlockSpec((bn, 128), lambda i: (i, 0)),
            pl.BlockSpec((1, 384, 128), lambda i: (i // bph, 0, 0)),
            pl.BlockSpec((1, 384, 128), lambda i: (i // bph, 0, 0)),
            pl.BlockSpec((1, 1, 384), lambda i: (i // bph, 0, 0)),
            pl.BlockSpec((1, 1, 384), lambda i: (i // bph, 0, 0)),
        ],
        out_specs=pl.BlockSpec((bn, 128), lambda i: (i, 0)),
        out_shape=jax.ShapeDtypeStruct((n, 128), F32),
    )(accs, state_prev, wih_s, whh_s, bih_s, bhh_s)


def kernel(msg_node, J_msg, b, state_prev, idx_msg_edge, node_idx,
           node_idx_inv, W1, b1, W2, b2, W3, b3, Wih1, Whh1, bih1, bhh1,
           Wih2, Whh2, bih2, bhh2):
    n, h = state_prev.shape
    e = msg_node.shape[0]
    del idx_msg_edge, node_idx, node_idx_inv

    wcat = jnp.concatenate([W1[:, :h], W1[:, h + 4:2 * h + 4]], axis=0)
    u = (W1[:, h] - W1[:, h + 1]).reshape(1, 64)
    v = (W1[:, 2 * h + 5] - W1[:, 2 * h + 4]).reshape(1, 64)
    uv = jnp.concatenate([u, v], axis=1)
    wj = (W1[:, h + 2] - W1[:, h + 3]
          + W1[:, 2 * h + 7] - W1[:, 2 * h + 6]).reshape(1, 64)
    b1cat = jnp.concatenate([b1.reshape(1, 64), jnp.zeros((1, 64), F32)],
                            axis=1)
    b2r = b2.reshape(1, 64)
    b3r = b3.reshape(1, 128)
    wih_s = jnp.stack([Wih1, Wih2])
    whh_s = jnp.stack([Whh1, Whh2])
    bih_s = jnp.stack([bih1, bih2]).reshape(2, 1, 384)
    bhh_s = jnp.stack([bhh1, bhh2]).reshape(2, 1, 384)
    src3d = msg_node[:, 0].reshape(e // 128, 1, 128)
    dst3d = msg_node[:, 1].reshape(e // 128, 1, 128)
    n_pad = ((n + 1279) // 1280) * 1280
    zeros_blk = jnp.zeros((n_pad // 16, 128), F32)

    pcat = _node_projections(state_prev, b, wcat, uv, b1cat)
    x1, x2 = _gather_ab(pcat, src3d, dst3d)
    msg = _edge_mlp(x1, x2, J_msg, wj, W2, b2r, W3, b3r)
    accs = _scatter_acc(msg, dst3d, zeros_blk, n_pad)
    return _gru_update(accs, state_prev, wih_s, whh_s, bih_s, bhh_s)


# speedup vs baseline: 1.3748x; 1.0692x over previous
import functools

import jax
import jax.numpy as jnp
from jax import lax
from jax.experimental import pallas as pl
from jax.experimental.pallas import tpu as pltpu
from jax.experimental.pallas import tpu_sc as plsc

F32 = jnp.float32


def _k1_body(state_ref, b_ref, wcat_ref, uv_ref, b1cat_ref, pcat_ref):
    dn = (((1,), (1,)), ((), ()))
    pcat_ref[...] = (lax.dot_general(state_ref[...], wcat_ref[...], dn,
                                     preferred_element_type=F32)
                     + b_ref[...] * uv_ref[...] + b1cat_ref[...])


def _node_projections(state_prev, b, wcat, uv, b1cat):
    n = state_prev.shape[0]
    return pl.pallas_call(
        _k1_body,
        out_shape=jax.ShapeDtypeStruct((n, 128), F32),
    )(state_prev, b, wcat, uv, b1cat)


def _gather_ab(pcat, src3d, dst3d):
    nblk = src3d.shape[0]
    w = 128
    e = nblk * w
    mesh = plsc.VectorSubcoreMesh(core_axis_name="core",
                                  subcore_axis_name="subcore")

    @functools.partial(
        pl.kernel,
        out_type=(jax.ShapeDtypeStruct((e, 128), F32),
                  jax.ShapeDtypeStruct((e, 128), F32)),
        mesh=mesh,
        scratch_types=[pltpu.SemaphoreType.DMA, pltpu.SemaphoreType.DMA],
    )
    def k2(pcat_hbm, src_hbm, dst_hbm, a_hbm, b_hbm, sem1, sem2):
        def body(si_vmem, di_vmem, a_vmem, b_vmem):
            c1 = pltpu.async_copy(pcat_hbm.at[si_vmem.at[0, 0]], a_vmem, sem1)
            c2 = pltpu.async_copy(pcat_hbm.at[di_vmem.at[0, 0]], b_vmem, sem2)
            c1.wait()
            c2.wait()

        pltpu.emit_pipeline(
            body,
            grid=(nblk,),
            in_specs=[pl.BlockSpec((1, 1, w), lambda i: (i, 0, 0)),
                      pl.BlockSpec((1, 1, w), lambda i: (i, 0, 0))],
            out_specs=[pl.BlockSpec((w, 128), lambda i: (i, 0)),
                       pl.BlockSpec((w, 128), lambda i: (i, 0))],
            core_axis_name=("core", "subcore"),
            dimension_semantics=(pltpu.PARALLEL,),
        )(src_hbm, dst_hbm, a_hbm, b_hbm)

    return k2(pcat, src3d, dst3d)


def _k3_body(x1_ref, x2_ref, j_ref, wj_ref, w2_ref, b2_ref, w3_ref, b3_ref,
             out_ref):
    dn = (((1,), (1,)), ((), ()))
    jterm = lax.dot_general(j_ref[...], wj_ref[...], (((1,), (0,)), ((), ())),
                            preferred_element_type=F32)
    x = jnp.maximum(x1_ref[:, 0:64] + x2_ref[:, 64:128] + jterm, 0.0)
    y = lax.dot_general(x, w2_ref[...], dn, preferred_element_type=F32)
    y = jnp.maximum(y + b2_ref[...], 0.0)
    out_ref[...] = (lax.dot_general(y, w3_ref[...], dn,
                                    preferred_element_type=F32)
                    + b3_ref[...])


def _edge_mlp(x1, x2, j_msg, wj, w2, b2r, w3, b3r):
    e = x1.shape[0]
    be = 4000
    grid = (e // be,)
    return pl.pallas_call(
        _k3_body,
        grid=grid,
        in_specs=[
            pl.BlockSpec((be, 128), lambda i: (i, 0)),
            pl.BlockSpec((be, 128), lambda i: (i, 0)),
            pl.BlockSpec((be, 1), lambda i: (i, 0)),
            pl.BlockSpec((1, 64), lambda i: (0, 0)),
            pl.BlockSpec((64, 64), lambda i: (0, 0)),
            pl.BlockSpec((1, 64), lambda i: (0, 0)),
            pl.BlockSpec((128, 64), lambda i: (0, 0)),
            pl.BlockSpec((1, 128), lambda i: (0, 0)),
        ],
        out_specs=pl.BlockSpec((be, 128), lambda i: (i, 0)),
        out_shape=jax.ShapeDtypeStruct((e, 128), F32),
    )(x1, x2, j_msg, wj, w2, b2r, w3, b3r)


def _scatter_acc(r, dst3d, zeros_blk, n_pad):
    e = r.shape[0]
    w = 128
    nblk = e // w
    rows_per_tile = n_pad // 16
    mesh = plsc.VectorSubcoreMesh(core_axis_name="core",
                                  subcore_axis_name="subcore")

    @functools.partial(
        pl.kernel,
        out_type=jax.ShapeDtypeStruct((2, n_pad, 128), F32),
        mesh=mesh,
        scratch_types=[
            pltpu.VMEM_SHARED((n_pad, 128), F32),
        ],
    )
    def k4(r_hbm, dst_hbm, z_hbm, out_hbm, acc_sp):
        cid = lax.axis_index("core")
        sid = lax.axis_index("subcore")
        row0 = sid * rows_per_tile

        pltpu.sync_copy(z_hbm, acc_sp.at[pl.ds(row0, rows_per_tile)])
        plsc.subcore_barrier()

        def body(r_vmem, di_vmem):
            pltpu.sync_copy(r_vmem, acc_sp.at[di_vmem.at[0, 0]], add=True)

        pltpu.emit_pipeline(
            body,
            grid=(nblk,),
            in_specs=[pl.BlockSpec((w, 128), lambda i: (i, 0)),
                      pl.BlockSpec((1, 1, w), lambda i: (i, 0, 0))],
            out_specs=[],
            core_axis_name=("core", "subcore"),
            dimension_semantics=(pltpu.PARALLEL,),
        )(r_hbm, dst_hbm)

        plsc.subcore_barrier()
        pltpu.sync_copy(acc_sp.at[pl.ds(row0, rows_per_tile)],
                        out_hbm.at[cid, pl.ds(row0, rows_per_tile)])

    return k4(r, dst3d, zeros_blk)


def _k5_body(accs_ref, state_ref, wih_ref, whh_ref, bih_ref, bhh_ref,
             out_ref):
    x = accs_ref[0] + accs_ref[1]
    for k in range(2, accs_ref.shape[0]):
        x = x + accs_ref[k]
    h = state_ref[...]
    dn = (((1,), (1,)), ((), ()))
    gx = lax.dot_general(x, wih_ref[0], dn, preferred_element_type=F32) \
        + bih_ref[0]
    gh = lax.dot_general(h, whh_ref[0], dn, preferred_element_type=F32) \
        + bhh_ref[0]
    d = 128
    rg = jax.nn.sigmoid(gx[:, :d] + gh[:, :d])
    zg = jax.nn.sigmoid(gx[:, d:2 * d] + gh[:, d:2 * d])
    ng = jnp.tanh(gx[:, 2 * d:] + rg * gh[:, 2 * d:])
    out_ref[...] = (1.0 - zg) * ng + zg * h


def _gru_update(accs, state_prev, wih_s, whh_s, bih_s, bhh_s):
    n = state_prev.shape[0]
    nacc = accs.shape[0]
    bn = 1000
    half = n // 2
    bph = half // bn
    grid = (n // bn,)
    return pl.pallas_call(
        _k5_body,
        grid=grid,
        in_specs=[
            pl.BlockSpec((nacc, bn, 128), lambda i: (0, i, 0)),
            pl.BlockSpec((bn, 128), lambda i: (i, 0)),
            pl.BlockSpec((1, 384, 128), lambda i: (i // bph, 0, 0)),
            pl.BlockSpec((1, 384, 128), lambda i: (i // bph, 0, 0)),
            pl.BlockSpec((1, 1, 384), lambda i: (i // bph, 0, 0)),
            pl.BlockSpec((1, 1, 384), lambda i: (i // bph, 0, 0)),
        ],
        out_specs=pl.BlockSpec((bn, 128), lambda i: (i, 0)),
        out_shape=jax.ShapeDtypeStruct((n, 128), F32),
    )(accs, state_prev, wih_s, whh_s, bih_s, bhh_s)


def kernel(msg_node, J_msg, b, state_prev, idx_msg_edge, node_idx,
           node_idx_inv, W1, b1, W2, b2, W3, b3, Wih1, Whh1, bih1, bhh1,
           Wih2, Whh2, bih2, bhh2):
    n, h = state_prev.shape
    e = msg_node.shape[0]
    del idx_msg_edge, node_idx, node_idx_inv

    wcat = jnp.concatenate([W1[:, :h], W1[:, h + 4:2 * h + 4]], axis=0)
    u = (W1[:, h] - W1[:, h + 1]).reshape(1, 64)
    v = (W1[:, 2 * h + 5] - W1[:, 2 * h + 4]).reshape(1, 64)
    uv = jnp.concatenate([u, v], axis=1)
    wj = (W1[:, h + 2] - W1[:, h + 3]
          + W1[:, 2 * h + 7] - W1[:, 2 * h + 6]).reshape(1, 64)
    b1cat = jnp.concatenate([b1.reshape(1, 64), jnp.zeros((1, 64), F32)],
                            axis=1)
    b2r = b2.reshape(1, 64)
    b3r = b3.reshape(1, 128)
    wih_s = jnp.stack([Wih1, Wih2])
    whh_s = jnp.stack([Whh1, Whh2])
    bih_s = jnp.stack([bih1, bih2]).reshape(2, 1, 384)
    bhh_s = jnp.stack([bhh1, bhh2]).reshape(2, 1, 384)
    src3d = msg_node[:, 0].reshape(e // 128, 1, 128)
    dst3d = msg_node[:, 1].reshape(e // 128, 1, 128)
    n_pad = ((n + 1279) // 1280) * 1280
    zeros_blk = jnp.zeros((n_pad // 16, 128), F32)

    pcat = _node_projections(state_prev, b, wcat, uv, b1cat)
    x1, x2 = _gather_ab(pcat, src3d, dst3d)
    msg = _edge_mlp(x1, x2, J_msg, wj, W2, b2r, W3, b3r)
    accs = _scatter_acc(msg, dst3d, zeros_blk, n_pad)
    return _gru_update(accs, state_prev, wih_s, whh_s, bih_s, bhh_s)


# speedup vs baseline: 1.3829x; 1.0059x over previous
import functools

import jax
import jax.numpy as jnp
from jax import lax
from jax.experimental import pallas as pl
from jax.experimental.pallas import tpu as pltpu
from jax.experimental.pallas import tpu_sc as plsc

F32 = jnp.float32


def _k1_body(state_ref, b_ref, wcat_ref, uv_ref, b1cat_ref, pcat_ref):
    dn = (((1,), (1,)), ((), ()))
    pcat_ref[...] = (lax.dot_general(state_ref[...], wcat_ref[...], dn,
                                     preferred_element_type=F32)
                     + b_ref[...] * uv_ref[...] + b1cat_ref[...])


def _node_projections(state_prev, b, wcat, uv, b1cat):
    n = state_prev.shape[0]
    return pl.pallas_call(
        _k1_body,
        out_shape=jax.ShapeDtypeStruct((n, 128), F32),
    )(state_prev, b, wcat, uv, b1cat)


def _gather_ab(pcat, src3d, dst3d):
    nblk = src3d.shape[0]
    w = 128
    e = nblk * w
    mesh = plsc.VectorSubcoreMesh(core_axis_name="core",
                                  subcore_axis_name="subcore")

    @functools.partial(
        pl.kernel,
        out_type=(jax.ShapeDtypeStruct((e, 128), F32),
                  jax.ShapeDtypeStruct((e, 128), F32)),
        mesh=mesh,
        scratch_types=[pltpu.SemaphoreType.DMA, pltpu.SemaphoreType.DMA],
    )
    def k2(pcat_hbm, src_hbm, dst_hbm, a_hbm, b_hbm, sem1, sem2):
        def body(si_vmem, di_vmem, a_vmem, b_vmem):
            c1 = pltpu.async_copy(pcat_hbm.at[si_vmem.at[0, 0]], a_vmem, sem1)
            c2 = pltpu.async_copy(pcat_hbm.at[di_vmem.at[0, 0]], b_vmem, sem2)
            c1.wait()
            c2.wait()

        pltpu.emit_pipeline(
            body,
            grid=(nblk,),
            in_specs=[pl.BlockSpec((1, 1, w), lambda i: (i, 0, 0)),
                      pl.BlockSpec((1, 1, w), lambda i: (i, 0, 0))],
            out_specs=[pl.BlockSpec((w, 128), lambda i: (i, 0)),
                       pl.BlockSpec((w, 128), lambda i: (i, 0))],
            core_axis_name=("core", "subcore"),
            dimension_semantics=(pltpu.PARALLEL,),
        )(src_hbm, dst_hbm, a_hbm, b_hbm)

    return k2(pcat, src3d, dst3d)


def _k3_body(x1_ref, x2_ref, j_ref, wj_ref, w2_ref, b2_ref, w3_ref, b3_ref,
             out_ref):
    dn = (((1,), (1,)), ((), ()))
    jterm = lax.dot_general(j_ref[...], wj_ref[...], (((1,), (0,)), ((), ())),
                            preferred_element_type=F32)
    x = jnp.maximum(x1_ref[:, 0:64] + x2_ref[:, 64:128] + jterm, 0.0)
    y = lax.dot_general(x, w2_ref[...], dn, preferred_element_type=F32)
    y = jnp.maximum(y + b2_ref[...], 0.0)
    out_ref[...] = (lax.dot_general(y, w3_ref[...], dn,
                                    preferred_element_type=F32)
                    + b3_ref[...])


def _edge_mlp(x1, x2, j_msg, wj, w2, b2r, w3, b3r):
    e = x1.shape[0]
    be = 8000
    grid = (e // be,)
    return pl.pallas_call(
        _k3_body,
        grid=grid,
        in_specs=[
            pl.BlockSpec((be, 128), lambda i: (i, 0)),
            pl.BlockSpec((be, 128), lambda i: (i, 0)),
            pl.BlockSpec((be, 1), lambda i: (i, 0)),
            pl.BlockSpec((1, 64), lambda i: (0, 0)),
            pl.BlockSpec((64, 64), lambda i: (0, 0)),
            pl.BlockSpec((1, 64), lambda i: (0, 0)),
            pl.BlockSpec((128, 64), lambda i: (0, 0)),
            pl.BlockSpec((1, 128), lambda i: (0, 0)),
        ],
        out_specs=pl.BlockSpec((be, 128), lambda i: (i, 0)),
        out_shape=jax.ShapeDtypeStruct((e, 128), F32),
    )(x1, x2, j_msg, wj, w2, b2r, w3, b3r)


def _scatter_acc(r, dst3d, zeros_blk, n_pad):
    e = r.shape[0]
    w = 128
    nblk = e // w
    rows_per_tile = n_pad // 16
    mesh = plsc.VectorSubcoreMesh(core_axis_name="core",
                                  subcore_axis_name="subcore")

    @functools.partial(
        pl.kernel,
        out_type=jax.ShapeDtypeStruct((2, n_pad, 128), F32),
        mesh=mesh,
        scratch_types=[
            pltpu.VMEM_SHARED((n_pad, 128), F32),
        ],
    )
    def k4(r_hbm, dst_hbm, z_hbm, out_hbm, acc_sp):
        cid = lax.axis_index("core")
        sid = lax.axis_index("subcore")
        row0 = sid * rows_per_tile

        pltpu.sync_copy(z_hbm, acc_sp.at[pl.ds(row0, rows_per_tile)])
        plsc.subcore_barrier()

        def body(r_vmem, di_vmem):
            pltpu.sync_copy(r_vmem, acc_sp.at[di_vmem.at[0, 0]], add=True)

        pltpu.emit_pipeline(
            body,
            grid=(nblk,),
            in_specs=[pl.BlockSpec((w, 128), lambda i: (i, 0)),
                      pl.BlockSpec((1, 1, w), lambda i: (i, 0, 0))],
            out_specs=[],
            core_axis_name=("core", "subcore"),
            dimension_semantics=(pltpu.PARALLEL,),
        )(r_hbm, dst_hbm)

        plsc.subcore_barrier()
        pltpu.sync_copy(acc_sp.at[pl.ds(row0, rows_per_tile)],
                        out_hbm.at[cid, pl.ds(row0, rows_per_tile)])

    return k4(r, dst3d, zeros_blk)


def _k5_body(accs_ref, state_ref, wih_ref, whh_ref, bih_ref, bhh_ref,
             out_ref):
    x = accs_ref[0] + accs_ref[1]
    for k in range(2, accs_ref.shape[0]):
        x = x + accs_ref[k]
    h = state_ref[...]
    dn = (((1,), (1,)), ((), ()))
    gx = lax.dot_general(x, wih_ref[0], dn, preferred_element_type=F32) \
        + bih_ref[0]
    gh = lax.dot_general(h, whh_ref[0], dn, preferred_element_type=F32) \
        + bhh_ref[0]
    d = 128
    rg = jax.nn.sigmoid(gx[:, :d] + gh[:, :d])
    zg = jax.nn.sigmoid(gx[:, d:2 * d] + gh[:, d:2 * d])
    ng = jnp.tanh(gx[:, 2 * d:] + rg * gh[:, 2 * d:])
    out_ref[...] = (1.0 - zg) * ng + zg * h


def _gru_update(accs, state_prev, wih_s, whh_s, bih_s, bhh_s):
    n = state_prev.shape[0]
    nacc = accs.shape[0]
    bn = 1000
    half = n // 2
    bph = half // bn
    grid = (n // bn,)
    return pl.pallas_call(
        _k5_body,
        grid=grid,
        in_specs=[
            pl.BlockSpec((nacc, bn, 128), lambda i: (0, i, 0)),
            pl.BlockSpec((bn, 128), lambda i: (i, 0)),
            pl.BlockSpec((1, 384, 128), lambda i: (i // bph, 0, 0)),
            pl.BlockSpec((1, 384, 128), lambda i: (i // bph, 0, 0)),
            pl.BlockSpec((1, 1, 384), lambda i: (i // bph, 0, 0)),
            pl.BlockSpec((1, 1, 384), lambda i: (i // bph, 0, 0)),
        ],
        out_specs=pl.BlockSpec((bn, 128), lambda i: (i, 0)),
        out_shape=jax.ShapeDtypeStruct((n, 128), F32),
    )(accs, state_prev, wih_s, whh_s, bih_s, bhh_s)


def kernel(msg_node, J_msg, b, state_prev, idx_msg_edge, node_idx,
           node_idx_inv, W1, b1, W2, b2, W3, b3, Wih1, Whh1, bih1, bhh1,
           Wih2, Whh2, bih2, bhh2):
    n, h = state_prev.shape
    e = msg_node.shape[0]
    del idx_msg_edge, node_idx, node_idx_inv

    wcat = jnp.concatenate([W1[:, :h], W1[:, h + 4:2 * h + 4]], axis=0)
    u = (W1[:, h] - W1[:, h + 1]).reshape(1, 64)
    v = (W1[:, 2 * h + 5] - W1[:, 2 * h + 4]).reshape(1, 64)
    uv = jnp.concatenate([u, v], axis=1)
    wj = (W1[:, h + 2] - W1[:, h + 3]
          + W1[:, 2 * h + 7] - W1[:, 2 * h + 6]).reshape(1, 64)
    b1cat = jnp.concatenate([b1.reshape(1, 64), jnp.zeros((1, 64), F32)],
                            axis=1)
    b2r = b2.reshape(1, 64)
    b3r = b3.reshape(1, 128)
    wih_s = jnp.stack([Wih1, Wih2])
    whh_s = jnp.stack([Whh1, Whh2])
    bih_s = jnp.stack([bih1, bih2]).reshape(2, 1, 384)
    bhh_s = jnp.stack([bhh1, bhh2]).reshape(2, 1, 384)
    src3d = msg_node[:, 0].reshape(e // 128, 1, 128)
    dst3d = msg_node[:, 1].reshape(e // 128, 1, 128)
    n_pad = ((n + 1279) // 1280) * 1280
    zeros_blk = jnp.zeros((n_pad // 16, 128), F32)

    pcat = _node_projections(state_prev, b, wcat, uv, b1cat)
    x1, x2 = _gather_ab(pcat, src3d, dst3d)
    msg = _edge_mlp(x1, x2, J_msg, wj, W2, b2r, W3, b3r)
    accs = _scatter_acc(msg, dst3d, zeros_blk, n_pad)
    return _gru_update(accs, state_prev, wih_s, whh_s, bih_s, bhh_s)
